# baseline (speedup 1.0000x reference)
import jax
import jax.numpy as jnp
from jax import lax
from jax.experimental import pallas as pl
from jax.experimental.pallas import tpu as pltpu
from jax.experimental.pallas import tpu_sc as plsc

SEQ_LEN = 2048
N_EMBD = 768
NUM_CORES = 2
NUM_SUBCORES = 16
NUM_WORKERS = NUM_CORES * NUM_SUBCORES
ROWS_PER_WORKER = SEQ_LEN // NUM_WORKERS
LANES = 16
VECS_PER_ROW = N_EMBD // LANES


def _emb_body(ids_hbm, wte_hbm, wpe_hbm, out_hbm, idx_v, ident_v, rows_v,
              wpe_v, gat_sem, lin_sem):
    wid = lax.axis_index("s") * NUM_CORES + lax.axis_index("c")
    base = wid * ROWS_PER_WORKER

    for k in range(ROWS_PER_WORKER // LANES):
        ident_v[pl.ds(k * LANES, LANES)] = lax.iota(jnp.int32, LANES) + (
            k * LANES)

    pltpu.sync_copy(ids_hbm.at[pl.ds(base, ROWS_PER_WORKER)], idx_v)

    gat = pltpu.async_copy(wte_hbm.at[idx_v], rows_v, gat_sem)
    lin = pltpu.async_copy(wpe_hbm.at[pl.ds(base, ROWS_PER_WORKER)], wpe_v,
                           lin_sem)
    gat.wait()
    lin.wait()

    CHUNK = 16
    NCH = ROWS_PER_WORKER // CHUNK

    def out_copy(k):
        return pltpu.make_async_copy(
            rows_v.at[pl.ds(k * CHUNK, CHUNK)],
            out_hbm.at[0, pl.ds(base + k * CHUNK, CHUNK)], gat_sem)

    for k in range(NCH):
        def add_row(j, carry, k=k):
            for i in range(VECS_PER_ROW):
                sl = pl.ds(i * LANES, LANES)
                rows_v[k * CHUNK + j, sl] += wpe_v[k * CHUNK + j, sl]
            return carry

        lax.fori_loop(0, CHUNK, add_row, 0, unroll=False)
        out_copy(k).start()

    for k in range(NCH):
        out_copy(k).wait()


@jax.jit
def _embedding(input_ids, wte, wpe):
    mesh = plsc.VectorSubcoreMesh(core_axis_name="c", subcore_axis_name="s")
    run = pl.kernel(
        _emb_body,
        out_type=jax.ShapeDtypeStruct((1, SEQ_LEN, N_EMBD), jnp.float32),
        mesh=mesh,
        scratch_types=[
            pltpu.VMEM((ROWS_PER_WORKER,), jnp.int32),
            pltpu.VMEM((ROWS_PER_WORKER,), jnp.int32),
            pltpu.VMEM((ROWS_PER_WORKER, N_EMBD), jnp.float32),
            pltpu.VMEM((ROWS_PER_WORKER, N_EMBD), jnp.float32),
            pltpu.SemaphoreType.DMA,
            pltpu.SemaphoreType.DMA,
        ],
    )
    return run(input_ids, wte, wpe)


def kernel(input_ids, wte, wpe):
    return _embedding(input_ids.astype(jnp.int32), wte, wpe)


# speedup vs baseline: 1.1890x; 1.1890x over previous
import jax
import jax.numpy as jnp
from jax import lax
from jax.experimental import pallas as pl
from jax.experimental.pallas import tpu as pltpu
from jax.experimental.pallas import tpu_sc as plsc

SEQ_LEN = 2048
N_EMBD = 768
NUM_CORES = 2
NUM_SUBCORES = 16
NUM_WORKERS = NUM_CORES * NUM_SUBCORES
ROWS_PER_WORKER = SEQ_LEN // NUM_WORKERS
LANES = 16
VECS_PER_ROW = N_EMBD // LANES


def _emb_body(ids_hbm, wte_hbm, wpe_hbm, out_hbm, idx_v, ident_v, rows_v,
              wpe_v, gat_sem, lin_sem):
    wid = lax.axis_index("s") * NUM_CORES + lax.axis_index("c")
    base = wid * ROWS_PER_WORKER

    for k in range(ROWS_PER_WORKER // LANES):
        ident_v[pl.ds(k * LANES, LANES)] = lax.iota(jnp.int32, LANES) + (
            k * LANES)

    lin = pltpu.async_copy(wpe_hbm.at[pl.ds(base, ROWS_PER_WORKER)], wpe_v,
                           lin_sem)
    pltpu.sync_copy(ids_hbm.at[pl.ds(base, ROWS_PER_WORKER)], idx_v)
    gat = pltpu.async_copy(wte_hbm.at[idx_v], rows_v, gat_sem)
    gat.wait()
    lin.wait()

    def add_row(j, carry):
        for i in range(VECS_PER_ROW):
            sl = pl.ds(i * LANES, LANES)
            rows_v[j, sl] += wpe_v[j, sl]
        return carry

    lax.fori_loop(0, ROWS_PER_WORKER, add_row, 0, unroll=False)

    pltpu.sync_copy(rows_v, out_hbm.at[0, pl.ds(base, ROWS_PER_WORKER)])


@jax.jit
def _embedding(input_ids, wte, wpe):
    mesh = plsc.VectorSubcoreMesh(core_axis_name="c", subcore_axis_name="s")
    run = pl.kernel(
        _emb_body,
        out_type=jax.ShapeDtypeStruct((1, SEQ_LEN, N_EMBD), jnp.float32),
        mesh=mesh,
        scratch_types=[
            pltpu.VMEM((ROWS_PER_WORKER,), jnp.int32),
            pltpu.VMEM((ROWS_PER_WORKER,), jnp.int32),
            pltpu.VMEM((ROWS_PER_WORKER, N_EMBD), jnp.float32),
            pltpu.VMEM((ROWS_PER_WORKER, N_EMBD), jnp.float32),
            pltpu.SemaphoreType.DMA,
            pltpu.SemaphoreType.DMA,
        ],
    )
    return run(input_ids, wte, wpe)


def kernel(input_ids, wte, wpe):
    return _embedding(input_ids.astype(jnp.int32), wte, wpe)
